# baseline (speedup 1.0000x reference)
import functools

import jax
import jax.numpy as jnp
from jax import lax
from jax.experimental import pallas as pl
from jax.experimental.pallas import tpu as pltpu
from jax.experimental.pallas import tpu_sc as plsc

BATCH = 16384
EMBED = 64
NROWS = 1000000

_NC, _NS = 2, 16
_NW = _NC * _NS
_BPW = BATCH // _NW


_TBLK = 8192
_QUAD = _TBLK // 4
_SHIFT = _TBLK.bit_length() - 1
_NSTG = (NROWS + _TBLK - 1) // _TBLK * _QUAD


def _rtne_bf16_bits(t):
    u = lax.bitcast_convert_type(t, jnp.uint32)
    rounded = (u + jnp.uint32(0x7FFF)
               + (lax.shift_right_logical(u, jnp.uint32(16)) & jnp.uint32(1)))
    return rounded & jnp.uint32(0xFFFF0000)


def _transpose_body(tablet_ref, out_ref):
    x = tablet_ref[...].astype(jnp.bfloat16)
    r = lax.broadcasted_iota(jnp.int32, (EMBED, EMBED), 0)
    c = lax.broadcasted_iota(jnp.int32, (EMBED, EMBED), 1)
    eye = (c == r).astype(jnp.bfloat16)
    q = [
        _rtne_bf16_bits(lax.dot_general(
            x[:, i * _QUAD:(i + 1) * _QUAD], eye, (((0,), (0,)), ((), ())),
            preferred_element_type=jnp.float32))
        for i in range(4)
    ]
    left = lax.bitcast_convert_type(
        q[0] | lax.shift_right_logical(q[1], jnp.uint32(16)), jnp.float32)
    right = lax.bitcast_convert_type(
        q[2] | lax.shift_right_logical(q[3], jnp.uint32(16)), jnp.float32)
    out_ref[...] = jnp.concatenate([left, right], axis=1)


def _transpose(tablet, *, interpret=False):
    grid = (pl.cdiv(NROWS, _TBLK),)
    return pl.pallas_call(
        _transpose_body,
        grid=grid,
        in_specs=[pl.BlockSpec((EMBED, _TBLK), lambda i: (0, i))],
        out_specs=pl.BlockSpec((_QUAD, 2 * EMBED), lambda i: (i, 0)),
        out_shape=jax.ShapeDtypeStruct((_NSTG, 2 * EMBED), jnp.float32),
        interpret=interpret,
    )(tablet)


@functools.cache
def _make_sc_gather():
    mesh = plsc.VectorSubcoreMesh(core_axis_name="c", subcore_axis_name="s")

    @functools.partial(
        pl.kernel,
        mesh=mesh,
        out_type=jax.ShapeDtypeStruct((BATCH, 2 * EMBED), jnp.float32),
        scratch_types=[
            pltpu.VMEM((_BPW,), jnp.int32),
            pltpu.VMEM((_BPW,), jnp.int32),
            pltpu.VMEM((_BPW, 2 * EMBED), jnp.float32),
            pltpu.SemaphoreType.DMA,
        ],
        compiler_params=pltpu.CompilerParams(use_tc_tiling_on_sc=False),
    )
    def _sc_gather(table_hbm, idx_hbm, out_hbm, idx_v, idxp_v, rows_v, sem):
        wid = lax.axis_index("s") * _NC + lax.axis_index("c")
        base = wid * _BPW
        pltpu.sync_copy(idx_hbm.at[pl.ds(base, _BPW)], idx_v)
        for g in range(_BPW // 16):
            r = idx_v[pl.ds(g * 16, 16)]
            idxp_v[pl.ds(g * 16, 16)] = (
                lax.shift_left(lax.shift_right_logical(r, _SHIFT), _SHIFT - 2)
                + (r & (_QUAD - 1)))
        pltpu.async_copy(table_hbm.at[idxp_v], rows_v, sem).wait()
        pltpu.sync_copy(rows_v, out_hbm.at[pl.ds(base, _BPW)])

    return _sc_gather


def _dense_body(uf_ref, rows_ref, ids_ref, W1_ref, b1_ref, W2_ref, b2_ref,
                Wc_ref, bc_ref, out_ref):
    h = jnp.maximum(
        jnp.dot(uf_ref[...], W1_ref[...], preferred_element_type=jnp.float32)
        + b1_ref[...], 0.0)
    feat = jnp.dot(h, W2_ref[...], preferred_element_type=jnp.float32) + b2_ref[...]
    rows = rows_ref[...]
    ids = ids_ref[...]
    right = (lax.shift_right_logical(ids, _SHIFT - 1) & 1) == 1
    lo = (lax.shift_right_logical(ids, _SHIFT - 2) & 1) == 1
    w = lax.bitcast_convert_type(
        jnp.where(right, rows[:, EMBED:], rows[:, :EMBED]), jnp.uint32)
    emb_bits = jnp.where(lo, lax.shift_left(w, jnp.uint32(16)),
                         w & jnp.uint32(0xFFFF0000))
    emb = lax.bitcast_convert_type(emb_bits, jnp.float32)
    Wc = Wc_ref[...]
    out = (jnp.dot(emb, Wc[:EMBED], preferred_element_type=jnp.float32)
           + jnp.dot(feat, Wc[EMBED:], preferred_element_type=jnp.float32)
           + bc_ref[...])
    norm = jnp.sqrt(jnp.sum(out * out, axis=-1, keepdims=True))
    out_ref[...] = out / jnp.maximum(norm, 1e-12)


def _dense(uf, rows, ids, W1, b1, W2, b2, Wc, bc, *, interpret=False):
    blk = 2048
    grid = (BATCH // blk,)
    nf = uf.shape[1]
    return pl.pallas_call(
        _dense_body,
        grid=grid,
        in_specs=[
            pl.BlockSpec((blk, nf), lambda i: (i, 0)),
            pl.BlockSpec((blk, 2 * EMBED), lambda i: (i, 0)),
            pl.BlockSpec((blk, 1), lambda i: (i, 0)),
            pl.BlockSpec((nf, 32), lambda i: (0, 0)),
            pl.BlockSpec((1, 32), lambda i: (0, 0)),
            pl.BlockSpec((32, EMBED), lambda i: (0, 0)),
            pl.BlockSpec((1, EMBED), lambda i: (0, 0)),
            pl.BlockSpec((2 * EMBED, EMBED), lambda i: (0, 0)),
            pl.BlockSpec((1, EMBED), lambda i: (0, 0)),
        ],
        out_specs=pl.BlockSpec((blk, EMBED), lambda i: (i, 0)),
        out_shape=jax.ShapeDtypeStruct((BATCH, EMBED), jnp.float32),
        interpret=interpret,
    )(uf, rows, ids, W1, b1, W2, b2, Wc, bc)


def kernel(user_ids, user_features, table, W1, b1, W2, b2, Wc, bc):
    ids = user_ids.astype(jnp.int32)
    table_pairs = _transpose(table.T)
    rows = _make_sc_gather()(table_pairs, ids)
    return _dense(user_features, rows, ids.reshape(-1, 1), W1,
                  b1.reshape(1, -1), W2, b2.reshape(1, -1), Wc,
                  bc.reshape(1, -1))


# speedup vs baseline: 1.1816x; 1.1816x over previous
import functools

import jax
import jax.numpy as jnp
from jax import lax
from jax.experimental import pallas as pl
from jax.experimental.pallas import tpu as pltpu
from jax.experimental.pallas import tpu_sc as plsc

BATCH = 16384
EMBED = 64
NROWS = 1000000

_NC, _NS = 2, 16
_NW = _NC * _NS
_BPW = BATCH // _NW


_TBLK = 16384
_HALF = _TBLK // 2
_SHIFT = _TBLK.bit_length() - 1
_NPAIR = (NROWS + _TBLK - 1) // _TBLK * _HALF


def _transpose_body(tablet_ref, out_ref):
    x = tablet_ref[...].astype(jnp.bfloat16)
    r = lax.broadcasted_iota(jnp.int32, (EMBED, 2 * EMBED), 0)
    c = lax.broadcasted_iota(jnp.int32, (EMBED, 2 * EMBED), 1)
    e1 = (c == r).astype(jnp.bfloat16)
    e2 = (c == r + EMBED).astype(jnp.bfloat16)
    out_ref[...] = (
        lax.dot_general(x[:, :_HALF], e1, (((0,), (0,)), ((), ())),
                        preferred_element_type=jnp.float32)
        + lax.dot_general(x[:, _HALF:], e2, (((0,), (0,)), ((), ())),
                          preferred_element_type=jnp.float32))


def _transpose(tablet, *, interpret=False):
    grid = (pl.cdiv(NROWS, _TBLK),)
    return pl.pallas_call(
        _transpose_body,
        grid=grid,
        in_specs=[pl.BlockSpec((EMBED, _TBLK), lambda i: (0, i))],
        out_specs=pl.BlockSpec((_HALF, 2 * EMBED), lambda i: (i, 0)),
        out_shape=jax.ShapeDtypeStruct((_NPAIR, 2 * EMBED), jnp.float32),
        interpret=interpret,
    )(tablet)


@functools.cache
def _make_sc_gather():
    mesh = plsc.VectorSubcoreMesh(core_axis_name="c", subcore_axis_name="s")

    @functools.partial(
        pl.kernel,
        mesh=mesh,
        out_type=jax.ShapeDtypeStruct((BATCH, 2 * EMBED), jnp.float32),
        scratch_types=[
            pltpu.VMEM((_BPW,), jnp.int32),
            pltpu.VMEM((_BPW,), jnp.int32),
            pltpu.VMEM((_BPW, 2 * EMBED), jnp.float32),
            pltpu.SemaphoreType.DMA,
        ],
        compiler_params=pltpu.CompilerParams(use_tc_tiling_on_sc=False),
    )
    def _sc_gather(table_hbm, idx_hbm, out_hbm, idx_v, idxp_v, rows_v, sem):
        wid = lax.axis_index("s") * _NC + lax.axis_index("c")
        base = wid * _BPW
        pltpu.sync_copy(idx_hbm.at[pl.ds(base, _BPW)], idx_v)
        for g in range(_BPW // 16):
            r = idx_v[pl.ds(g * 16, 16)]
            idxp_v[pl.ds(g * 16, 16)] = (
                lax.shift_left(lax.shift_right_logical(r, _SHIFT), _SHIFT - 1)
                + (r & (_HALF - 1)))
        pltpu.async_copy(table_hbm.at[idxp_v], rows_v, sem).wait()
        pltpu.sync_copy(rows_v, out_hbm.at[pl.ds(base, _BPW)])

    return _sc_gather


def _dense_body(uf_ref, rows_ref, ids_ref, W1_ref, b1_ref, W2_ref, b2_ref,
                Wc_ref, bc_ref, out_ref):
    h = jnp.maximum(
        jnp.dot(uf_ref[...], W1_ref[...], preferred_element_type=jnp.float32)
        + b1_ref[...], 0.0)
    feat = jnp.dot(h, W2_ref[...], preferred_element_type=jnp.float32) + b2_ref[...]
    rows = rows_ref[...]
    odd = (lax.shift_right_logical(ids_ref[...], _SHIFT - 1) & 1) == 1
    emb = jnp.where(odd, rows[:, EMBED:], rows[:, :EMBED])
    Wc = Wc_ref[...]
    out = (jnp.dot(emb, Wc[:EMBED], preferred_element_type=jnp.float32)
           + jnp.dot(feat, Wc[EMBED:], preferred_element_type=jnp.float32)
           + bc_ref[...])
    norm = jnp.sqrt(jnp.sum(out * out, axis=-1, keepdims=True))
    out_ref[...] = out / jnp.maximum(norm, 1e-12)


def _dense(uf, rows, ids, W1, b1, W2, b2, Wc, bc, *, interpret=False):
    blk = 2048
    grid = (BATCH // blk,)
    nf = uf.shape[1]
    return pl.pallas_call(
        _dense_body,
        grid=grid,
        in_specs=[
            pl.BlockSpec((blk, nf), lambda i: (i, 0)),
            pl.BlockSpec((blk, 2 * EMBED), lambda i: (i, 0)),
            pl.BlockSpec((blk, 1), lambda i: (i, 0)),
            pl.BlockSpec((nf, 32), lambda i: (0, 0)),
            pl.BlockSpec((1, 32), lambda i: (0, 0)),
            pl.BlockSpec((32, EMBED), lambda i: (0, 0)),
            pl.BlockSpec((1, EMBED), lambda i: (0, 0)),
            pl.BlockSpec((2 * EMBED, EMBED), lambda i: (0, 0)),
            pl.BlockSpec((1, EMBED), lambda i: (0, 0)),
        ],
        out_specs=pl.BlockSpec((blk, EMBED), lambda i: (i, 0)),
        out_shape=jax.ShapeDtypeStruct((BATCH, EMBED), jnp.float32),
        interpret=interpret,
    )(uf, rows, ids, W1, b1, W2, b2, Wc, bc)


def kernel(user_ids, user_features, table, W1, b1, W2, b2, Wc, bc):
    ids = user_ids.astype(jnp.int32)
    table_pairs = _transpose(table.T)
    rows = _make_sc_gather()(table_pairs, ids)
    return _dense(user_features, rows, ids.reshape(-1, 1), W1,
                  b1.reshape(1, -1), W2, b2.reshape(1, -1), Wc,
                  bc.reshape(1, -1))


# speedup vs baseline: 1.2133x; 1.0268x over previous
import functools

import jax
import jax.numpy as jnp
from jax import lax
from jax.experimental import pallas as pl
from jax.experimental.pallas import tpu as pltpu
from jax.experimental.pallas import tpu_sc as plsc

BATCH = 16384
EMBED = 64
NROWS = 1000000

_NC, _NS = 2, 16
_NW = _NC * _NS
_BPW = BATCH // _NW


_TBLK = 32768
_HALF = _TBLK // 2
_SHIFT = _TBLK.bit_length() - 1
_NPAIR = (NROWS + _TBLK - 1) // _TBLK * _HALF


def _transpose_body(tablet_ref, out_ref):
    x = tablet_ref[...].astype(jnp.bfloat16)
    r = lax.broadcasted_iota(jnp.int32, (EMBED, 2 * EMBED), 0)
    c = lax.broadcasted_iota(jnp.int32, (EMBED, 2 * EMBED), 1)
    e1 = (c == r).astype(jnp.bfloat16)
    e2 = (c == r + EMBED).astype(jnp.bfloat16)
    out_ref[...] = (
        lax.dot_general(x[:, :_HALF], e1, (((0,), (0,)), ((), ())),
                        preferred_element_type=jnp.float32)
        + lax.dot_general(x[:, _HALF:], e2, (((0,), (0,)), ((), ())),
                          preferred_element_type=jnp.float32))


def _transpose(tablet, *, interpret=False):
    grid = (pl.cdiv(NROWS, _TBLK),)
    return pl.pallas_call(
        _transpose_body,
        grid=grid,
        in_specs=[pl.BlockSpec((EMBED, _TBLK), lambda i: (0, i))],
        out_specs=pl.BlockSpec((_HALF, 2 * EMBED), lambda i: (i, 0)),
        out_shape=jax.ShapeDtypeStruct((_NPAIR, 2 * EMBED), jnp.float32),
        interpret=interpret,
    )(tablet)


@functools.cache
def _make_sc_gather():
    mesh = plsc.VectorSubcoreMesh(core_axis_name="c", subcore_axis_name="s")

    @functools.partial(
        pl.kernel,
        mesh=mesh,
        out_type=jax.ShapeDtypeStruct((BATCH, 2 * EMBED), jnp.float32),
        scratch_types=[
            pltpu.VMEM((_BPW,), jnp.int32),
            pltpu.VMEM((_BPW,), jnp.int32),
            pltpu.VMEM((_BPW, 2 * EMBED), jnp.float32),
            pltpu.SemaphoreType.DMA,
        ],
        compiler_params=pltpu.CompilerParams(use_tc_tiling_on_sc=False),
    )
    def _sc_gather(table_hbm, idx_hbm, out_hbm, idx_v, idxp_v, rows_v, sem):
        wid = lax.axis_index("s") * _NC + lax.axis_index("c")
        base = wid * _BPW
        pltpu.sync_copy(idx_hbm.at[pl.ds(base, _BPW)], idx_v)
        for g in range(_BPW // 16):
            r = idx_v[pl.ds(g * 16, 16)]
            idxp_v[pl.ds(g * 16, 16)] = (
                lax.shift_left(lax.shift_right_logical(r, _SHIFT), _SHIFT - 1)
                + (r & (_HALF - 1)))
        pltpu.async_copy(table_hbm.at[idxp_v], rows_v, sem).wait()
        pltpu.sync_copy(rows_v, out_hbm.at[pl.ds(base, _BPW)])

    return _sc_gather


def _dense_body(uf_ref, rows_ref, ids_ref, W1_ref, b1_ref, W2_ref, b2_ref,
                Wc_ref, bc_ref, out_ref):
    h = jnp.maximum(
        jnp.dot(uf_ref[...], W1_ref[...], preferred_element_type=jnp.float32)
        + b1_ref[...], 0.0)
    feat = jnp.dot(h, W2_ref[...], preferred_element_type=jnp.float32) + b2_ref[...]
    rows = rows_ref[...]
    odd = (lax.shift_right_logical(ids_ref[...], _SHIFT - 1) & 1) == 1
    emb = jnp.where(odd, rows[:, EMBED:], rows[:, :EMBED])
    Wc = Wc_ref[...]
    out = (jnp.dot(emb, Wc[:EMBED], preferred_element_type=jnp.float32)
           + jnp.dot(feat, Wc[EMBED:], preferred_element_type=jnp.float32)
           + bc_ref[...])
    norm = jnp.sqrt(jnp.sum(out * out, axis=-1, keepdims=True))
    out_ref[...] = out / jnp.maximum(norm, 1e-12)


def _dense(uf, rows, ids, W1, b1, W2, b2, Wc, bc, *, interpret=False):
    blk = 2048
    grid = (BATCH // blk,)
    nf = uf.shape[1]
    return pl.pallas_call(
        _dense_body,
        grid=grid,
        in_specs=[
            pl.BlockSpec((blk, nf), lambda i: (i, 0)),
            pl.BlockSpec((blk, 2 * EMBED), lambda i: (i, 0)),
            pl.BlockSpec((blk, 1), lambda i: (i, 0)),
            pl.BlockSpec((nf, 32), lambda i: (0, 0)),
            pl.BlockSpec((1, 32), lambda i: (0, 0)),
            pl.BlockSpec((32, EMBED), lambda i: (0, 0)),
            pl.BlockSpec((1, EMBED), lambda i: (0, 0)),
            pl.BlockSpec((2 * EMBED, EMBED), lambda i: (0, 0)),
            pl.BlockSpec((1, EMBED), lambda i: (0, 0)),
        ],
        out_specs=pl.BlockSpec((blk, EMBED), lambda i: (i, 0)),
        out_shape=jax.ShapeDtypeStruct((BATCH, EMBED), jnp.float32),
        interpret=interpret,
    )(uf, rows, ids, W1, b1, W2, b2, Wc, bc)


def kernel(user_ids, user_features, table, W1, b1, W2, b2, Wc, bc):
    ids = user_ids.astype(jnp.int32)
    table_pairs = _transpose(table.T)
    rows = _make_sc_gather()(table_pairs, ids)
    return _dense(user_features, rows, ids.reshape(-1, 1), W1,
                  b1.reshape(1, -1), W2, b2.reshape(1, -1), Wc,
                  bc.reshape(1, -1))


# speedup vs baseline: 1.2524x; 1.0323x over previous
import functools

import jax
import jax.numpy as jnp
from jax import lax
from jax.experimental import pallas as pl
from jax.experimental.pallas import tpu as pltpu
from jax.experimental.pallas import tpu_sc as plsc

BATCH = 16384
EMBED = 64
NROWS = 1000000

_NC, _NS = 2, 16
_NW = _NC * _NS
_BPW = BATCH // _NW


_TBLK = 32768
_QUAD = _TBLK // 4
_SHIFT = _TBLK.bit_length() - 1
_NSTG = (NROWS + _TBLK - 1) // _TBLK * _QUAD


def _rtne_bf16_bits(t):
    u = lax.bitcast_convert_type(t, jnp.uint32)
    rounded = (u + jnp.uint32(0x7FFF)
               + (lax.shift_right_logical(u, jnp.uint32(16)) & jnp.uint32(1)))
    return rounded & jnp.uint32(0xFFFF0000)


def _transpose_body(tablet_ref, out_ref):
    x = tablet_ref[...].astype(jnp.bfloat16)
    r = lax.broadcasted_iota(jnp.int32, (EMBED, EMBED), 0)
    c = lax.broadcasted_iota(jnp.int32, (EMBED, EMBED), 1)
    eye = (c == r).astype(jnp.bfloat16)
    q = [
        _rtne_bf16_bits(lax.dot_general(
            x[:, i * _QUAD:(i + 1) * _QUAD], eye, (((0,), (0,)), ((), ())),
            preferred_element_type=jnp.float32))
        for i in range(4)
    ]
    left = lax.bitcast_convert_type(
        q[0] | lax.shift_right_logical(q[1], jnp.uint32(16)), jnp.float32)
    right = lax.bitcast_convert_type(
        q[2] | lax.shift_right_logical(q[3], jnp.uint32(16)), jnp.float32)
    out_ref[...] = jnp.concatenate([left, right], axis=1)


def _transpose(tablet, *, interpret=False):
    grid = (pl.cdiv(NROWS, _TBLK),)
    return pl.pallas_call(
        _transpose_body,
        grid=grid,
        in_specs=[pl.BlockSpec((EMBED, _TBLK), lambda i: (0, i))],
        out_specs=pl.BlockSpec((_QUAD, 2 * EMBED), lambda i: (i, 0)),
        out_shape=jax.ShapeDtypeStruct((_NSTG, 2 * EMBED), jnp.float32),
        compiler_params=pltpu.CompilerParams(
            vmem_limit_bytes=112 * 1024 * 1024),
        interpret=interpret,
    )(tablet)


@functools.cache
def _make_sc_gather():
    mesh = plsc.VectorSubcoreMesh(core_axis_name="c", subcore_axis_name="s")

    @functools.partial(
        pl.kernel,
        mesh=mesh,
        out_type=jax.ShapeDtypeStruct((BATCH, 2 * EMBED), jnp.float32),
        scratch_types=[
            pltpu.VMEM((_BPW,), jnp.int32),
            pltpu.VMEM((_BPW,), jnp.int32),
            pltpu.VMEM((_BPW, 2 * EMBED), jnp.float32),
            pltpu.SemaphoreType.DMA,
        ],
        compiler_params=pltpu.CompilerParams(use_tc_tiling_on_sc=False),
    )
    def _sc_gather(table_hbm, idx_hbm, out_hbm, idx_v, idxp_v, rows_v, sem):
        wid = lax.axis_index("s") * _NC + lax.axis_index("c")
        base = wid * _BPW
        pltpu.sync_copy(idx_hbm.at[pl.ds(base, _BPW)], idx_v)
        for g in range(_BPW // 16):
            r = idx_v[pl.ds(g * 16, 16)]
            idxp_v[pl.ds(g * 16, 16)] = (
                lax.shift_left(lax.shift_right_logical(r, _SHIFT), _SHIFT - 2)
                + (r & (_QUAD - 1)))
        pltpu.async_copy(table_hbm.at[idxp_v], rows_v, sem).wait()
        pltpu.sync_copy(rows_v, out_hbm.at[pl.ds(base, _BPW)])

    return _sc_gather


def _dense_body(uf_ref, rows_ref, ids_ref, W1_ref, b1_ref, W2_ref, b2_ref,
                Wc_ref, bc_ref, out_ref):
    h = jnp.maximum(
        jnp.dot(uf_ref[...], W1_ref[...], preferred_element_type=jnp.float32)
        + b1_ref[...], 0.0)
    feat = jnp.dot(h, W2_ref[...], preferred_element_type=jnp.float32) + b2_ref[...]
    rows = rows_ref[...]
    ids = ids_ref[...]
    right = (lax.shift_right_logical(ids, _SHIFT - 1) & 1) == 1
    lo = (lax.shift_right_logical(ids, _SHIFT - 2) & 1) == 1
    w = lax.bitcast_convert_type(
        jnp.where(right, rows[:, EMBED:], rows[:, :EMBED]), jnp.uint32)
    emb_bits = jnp.where(lo, lax.shift_left(w, jnp.uint32(16)),
                         w & jnp.uint32(0xFFFF0000))
    emb = lax.bitcast_convert_type(emb_bits, jnp.float32)
    Wc = Wc_ref[...]
    out = (jnp.dot(emb, Wc[:EMBED], preferred_element_type=jnp.float32)
           + jnp.dot(feat, Wc[EMBED:], preferred_element_type=jnp.float32)
           + bc_ref[...])
    norm = jnp.sqrt(jnp.sum(out * out, axis=-1, keepdims=True))
    out_ref[...] = out / jnp.maximum(norm, 1e-12)


def _dense(uf, rows, ids, W1, b1, W2, b2, Wc, bc, *, interpret=False):
    blk = 2048
    grid = (BATCH // blk,)
    nf = uf.shape[1]
    return pl.pallas_call(
        _dense_body,
        grid=grid,
        in_specs=[
            pl.BlockSpec((blk, nf), lambda i: (i, 0)),
            pl.BlockSpec((blk, 2 * EMBED), lambda i: (i, 0)),
            pl.BlockSpec((blk, 1), lambda i: (i, 0)),
            pl.BlockSpec((nf, 32), lambda i: (0, 0)),
            pl.BlockSpec((1, 32), lambda i: (0, 0)),
            pl.BlockSpec((32, EMBED), lambda i: (0, 0)),
            pl.BlockSpec((1, EMBED), lambda i: (0, 0)),
            pl.BlockSpec((2 * EMBED, EMBED), lambda i: (0, 0)),
            pl.BlockSpec((1, EMBED), lambda i: (0, 0)),
        ],
        out_specs=pl.BlockSpec((blk, EMBED), lambda i: (i, 0)),
        out_shape=jax.ShapeDtypeStruct((BATCH, EMBED), jnp.float32),
        interpret=interpret,
    )(uf, rows, ids, W1, b1, W2, b2, Wc, bc)


def kernel(user_ids, user_features, table, W1, b1, W2, b2, Wc, bc):
    ids = user_ids.astype(jnp.int32)
    table_pairs = _transpose(table.T)
    rows = _make_sc_gather()(table_pairs, ids)
    return _dense(user_features, rows, ids.reshape(-1, 1), W1,
                  b1.reshape(1, -1), W2, b2.reshape(1, -1), Wc,
                  bc.reshape(1, -1))
